# baseline (speedup 1.0000x reference)
import functools

import jax
import jax.numpy as jnp
from jax import lax
from jax.experimental import pallas as pl
from jax.experimental.pallas import tpu as pltpu
from jax.experimental.pallas import tpu_sc as plsc

N = 10000
E = 160000
N_PAD = 10240
E_PAD = 163840
DUMP_DST = N_PAD
RANGE = 640
TILE_E = E_PAD // 32
HALF_E = E_PAD // 2

_mesh = plsc.VectorSubcoreMesh(core_axis_name="c", subcore_axis_name="s")


@functools.partial(
    pl.kernel,
    out_type=jax.ShapeDtypeStruct((E_PAD, 16), jnp.float32),
    mesh=_mesh,
    compiler_params=pltpu.CompilerParams(use_tc_tiling_on_sc=False,
                                         needs_layout_passes=False),
    scratch_types=[
        pltpu.VMEM((2, 512), jnp.int32),
        pltpu.VMEM((2, 512, 16), jnp.float32),
        pltpu.SemaphoreType.DMA,
        pltpu.SemaphoreType.DMA,
    ],
)
def _gather_x(src_hbm, xpad_hbm, out_hbm, idx_v, rows_v, s0, s1):
    wid = lax.axis_index("s") * 2 + lax.axis_index("c")
    base = wid * TILE_E
    sems = (s0, s1)

    def body(j, _):
        for q in range(2):
            b = base + (j * 2 + q) * 512
            pltpu.sync_copy(src_hbm.at[pl.ds(b, 512)], idx_v.at[q])
            hs = pltpu.async_copy(xpad_hbm.at[idx_v.at[q]], rows_v.at[q],
                                  sems[q])
            if q == 0:
                hs0 = hs
        hs0.wait()
        pltpu.sync_copy(rows_v.at[0],
                        out_hbm.at[pl.ds(base + j * 1024, 512)])
        hs.wait()
        pltpu.sync_copy(rows_v.at[1],
                        out_hbm.at[pl.ds(base + j * 1024 + 512, 512)])
        return 0

    lax.fori_loop(0, TILE_E // 1024, body, 0)


@functools.partial(
    pl.kernel,
    out_type=(
        jax.ShapeDtypeStruct((2, N_PAD, 128), jnp.float32),
        jax.ShapeDtypeStruct((2, N_PAD, 16), jnp.float32),
    ),
    mesh=_mesh,
    compiler_params=pltpu.CompilerParams(use_tc_tiling_on_sc=False,
                                         needs_layout_passes=False),
    scratch_types=[
        pltpu.VMEM((2, 128), jnp.int32),
        pltpu.VMEM((2, 128, 128), jnp.float32),
        pltpu.VMEM((128, 16), jnp.float32),
        pltpu.VMEM((128, 16), jnp.float32),
        pltpu.VMEM_SHARED((N_PAD + 128, 128), jnp.float32),
        pltpu.VMEM_SHARED((N_PAD + 128, 16), jnp.float32),
    ] + [pltpu.SemaphoreType.DMA] * 8,
)
def _segment_sum(dst_hbm, msg_hbm, aggp_hbm, degp_hbm,
                 idx_v, rows_v, ones_v, zd_v, agg_sh, deg_sh,
                 li0, li1, lm0, lm1, sa0, sa1, sd0, sd1):
    c = lax.axis_index("c")
    s = lax.axis_index("s")
    zero16 = jnp.zeros((16,), jnp.float32)
    one16 = jnp.ones((16,), jnp.float32)
    li = (li0, li1)
    lm = (lm0, lm1)
    sa = (sa0, sa1)
    sd = (sd0, sd1)

    def initrow(r, _):
        for j in range(8):
            rows_v[0, r, pl.ds(j * 16, 16)] = zero16
        ones_v[r, pl.ds(0, 16)] = one16
        zd_v[r, pl.ds(0, 16)] = zero16
        return 0

    lax.fori_loop(0, 128, initrow, 0)

    for k in range(RANGE // 128):
        pltpu.sync_copy(rows_v.at[0],
                        agg_sh.at[pl.ds(s * RANGE + k * 128, 128)])
        pltpu.sync_copy(zd_v, deg_sh.at[pl.ds(s * RANGE + k * 128, 128)])

    @pl.when(s == 0)
    def _():
        pltpu.sync_copy(rows_v.at[0], agg_sh.at[pl.ds(N_PAD, 128)])
        pltpu.sync_copy(zd_v, deg_sh.at[pl.ds(N_PAD, 128)])

    plsc.subcore_barrier()

    base = (s * 2 + c) * TILE_E

    def body(j2, _):
        hl = []
        for q in range(2):
            b = base + (j2 * 2 + q) * 128
            hl.append((pltpu.async_copy(dst_hbm.at[pl.ds(b, 128)],
                                        idx_v.at[q], li[q]),
                       pltpu.async_copy(msg_hbm.at[pl.ds(b, 128)],
                                        rows_v.at[q], lm[q])))
        hs = []
        for q in range(2):
            hl[q][0].wait()
            hl[q][1].wait()
            hs.append((pltpu.async_copy(rows_v.at[q],
                                        agg_sh.at[idx_v.at[q]], sa[q],
                                        add=True),
                       pltpu.async_copy(ones_v,
                                        deg_sh.at[idx_v.at[q]], sd[q],
                                        add=True)))
        for q in range(2):
            hs[q][0].wait()
            hs[q][1].wait()
        return 0

    lax.fori_loop(0, TILE_E // 256, body, 0)
    plsc.subcore_barrier()

    pltpu.sync_copy(agg_sh.at[pl.ds(s * RANGE, RANGE)],
                    aggp_hbm.at[c, pl.ds(s * RANGE, RANGE)])
    pltpu.sync_copy(deg_sh.at[pl.ds(s * RANGE, RANGE)],
                    degp_hbm.at[c, pl.ds(s * RANGE, RANGE)])


CAP = HALF_E + 512
CAPT = 32768
LBUF = 36864


@functools.partial(
    pl.kernel,
    out_type=(
        jax.ShapeDtypeStruct((32, CAP), jnp.int32),
        jax.ShapeDtypeStruct((32, CAP), jnp.int32),
        jax.ShapeDtypeStruct((32, 16), jnp.int32),
    ),
    mesh=_mesh,
    compiler_params=pltpu.CompilerParams(use_tc_tiling_on_sc=False,
                                         needs_layout_passes=False),
    scratch_types=[
        pltpu.VMEM((2, 2048), jnp.int32),
        pltpu.VMEM((2, 2048), jnp.int32),
        pltpu.VMEM((LBUF,), jnp.int32),
        pltpu.VMEM((LBUF,), jnp.int32),
        pltpu.VMEM((16,), jnp.int32),
    ] + [pltpu.SemaphoreType.DMA] * 4,
)
def _bucketize(dst_hbm, src_hbm, slist_hbm, dlist_hbm, cnt_hbm,
               dv, sv, sl, dl, cb, c0, c1, c2, c3):
    c = lax.axis_index("c")
    s = lax.axis_index("s")
    wid = s * 2 + c
    lo = s * RANGE
    zero16i = jnp.zeros((16,), jnp.int32)
    sent16 = jnp.full((16,), RANGE, jnp.int32)
    ebase = c * HALF_E

    def flush(nblk, woff, offbase):
        def cp(i, _):
            so = pl.multiple_of(offbase + i * 128, 128)
            ho = pl.multiple_of(woff + i * 128, 128)
            pltpu.sync_copy(sl.at[pl.ds(so, 128)],
                            slist_hbm.at[wid, pl.ds(ho, 128)])
            pltpu.sync_copy(dl.at[pl.ds(so, 128)],
                            dlist_hbm.at[wid, pl.ds(ho, 128)])
            return 0
        lax.fori_loop(0, nblk, cp, 0)

    csem = ((c0, c1), (c2, c3))

    def load_chunk(ch, p):
        cb2 = ebase + ch * 2048
        pltpu.async_copy(dst_hbm.at[pl.ds(cb2, 2048)], dv.at[p],
                         csem[p][0])
        pltpu.async_copy(src_hbm.at[pl.ds(cb2, 2048)], sv.at[p],
                         csem[p][1])

    def wait_chunk(p):
        pltpu.make_async_copy(dst_hbm.at[pl.ds(0, 2048)], dv.at[p],
                              csem[p][0]).wait()
        pltpu.make_async_copy(src_hbm.at[pl.ds(0, 2048)], sv.at[p],
                              csem[p][1]).wait()

    def scan_chunk(p, carry):
        off, woff = carry
        wait_chunk(p)

        def scan(v, o):
            dvec = dv[p, pl.ds(v * 16, 16)]
            svec = sv[p, pl.ds(v * 16, 16)]
            m = (dvec >= lo) & (dvec < lo + RANGE)
            pos = o + plsc.cumsum(m.astype(jnp.int32)) - 1
            plsc.store_scatter(sl, [pos], svec, mask=m)
            plsc.store_scatter(dl, [pos], dvec - lo, mask=m)
            return pos[15] + 1

        off = lax.fori_loop(0, 128, scan, off)

        nblk = jnp.where(off >= CAPT, off // 128, 0)
        flush(nblk, woff, 0)
        rem_base = nblk * 128
        for t in range(8):
            sl[pl.ds(t * 16, 16)] = sl[pl.ds(rem_base + t * 16, 16)]
            dl[pl.ds(t * 16, 16)] = dl[pl.ds(rem_base + t * 16, 16)]
        return off - rem_base, woff + rem_base

    npairs = HALF_E // 2048 // 2
    load_chunk(0, 0)

    def pair_body(kk, carry):
        ch0 = kk * 2
        load_chunk(ch0 + 1, 1)
        carry = scan_chunk(0, carry)

        @pl.when(kk < npairs - 1)
        def _():
            load_chunk(ch0 + 2, 0)
        return scan_chunk(1, carry)

    off, woff = lax.fori_loop(0, npairs, pair_body, (0, 0))

    for t in range(32):
        sl[pl.ds(off + t * 16, 16)] = zero16i
        dl[pl.ds(off + t * 16, 16)] = sent16
    total = woff + off
    total_p = ((total + 511) // 512) * 512
    off_p = total_p - woff
    flush(off_p // 128, woff, 0)
    cb[pl.ds(0, 16)] = jnp.full((16,), total_p, jnp.int32)
    pltpu.sync_copy(cb, cnt_hbm.at[wid])


def _make_segment_max(d):
    nvr = d // 16

    @functools.partial(
        pl.kernel,
        out_type=jax.ShapeDtypeStruct((2, N_PAD * d), jnp.float32),
        mesh=_mesh,
        compiler_params=pltpu.CompilerParams(use_tc_tiling_on_sc=False,
                                         needs_layout_passes=False),
        scratch_types=[
            pltpu.VMEM((2, 512), jnp.int32),
            pltpu.VMEM((2, 512), jnp.int32),
            pltpu.VMEM((16,), jnp.int32),
            pltpu.VMEM((2 if d <= 32 else 1, 4, 128, d), jnp.float32),
            pltpu.VMEM(((RANGE + 8) * d,), jnp.float32),
        ] + [pltpu.SemaphoreType.DMA] * 12,
    )
    def seg_max(slist_hbm, dlist_hbm, cnt_hbm, a_hbm, out_hbm,
                sbuf, dbuf, cb, rows_v, acc,
                s0, s1, s2, s3, s4, s5, s6, s7, l0, l1, l2, l3):
        c = lax.axis_index("c")
        s = lax.axis_index("s")
        wid = s * 2 + c
        lo = s * RANGE
        neg = jnp.full((16,), -3.0e38, jnp.float32)
        sems = ((s0, s1, s2, s3), (s4, s5, s6, s7))
        lsem = ((l0, l1), (l2, l3))

        def initrow(r, _):
            acc[pl.ds(r * 16, 16)] = neg
            return 0

        lax.fori_loop(0, (RANGE + 8) * d // 16, initrow, 0)

        pltpu.sync_copy(cnt_hbm.at[wid], cb)
        total = cb[pl.ds(0, 16)][0]
        ng = total // 512

        def load_lists(g, p):
            gb = g * 512
            return (pltpu.async_copy(slist_hbm.at[wid, pl.ds(gb, 512)],
                                     sbuf.at[p], lsem[p][0]),
                    pltpu.async_copy(dlist_hbm.at[wid, pl.ds(gb, 512)],
                                     dbuf.at[p], lsem[p][1]))

        nrb = 2 if d <= 32 else 1

        def gather_rows(p, hl):
            hl[0].wait()
            hl[1].wait()
            return [pltpu.async_copy(
                        a_hbm.at[sbuf.at[p, pl.ds(q * 128, 128)]],
                        rows_v.at[p % nrb, q], sems[p][q])
                    for q in range(4)]

        def rmw_group(p, hg):
            for q in range(4):
                hg[q].wait()

                def rmw(g8, _):
                    basev = dbuf[p, pl.ds(q * 128 + g8 * 16, 16)] * d
                    for b in range(16):
                        ab = basev[b]
                        for j in range(nvr):
                            cur = acc[pl.ds(ab + j * 16, 16)]
                            acc[pl.ds(ab + j * 16, 16)] = jnp.maximum(
                                cur,
                                rows_v[p % nrb, q,
                                       g8 * 16 + b, pl.ds(j * 16, 16)])
                    return 0

                lax.fori_loop(0, 8, rmw, 0)

        def pair(kk, _):
            g0 = kk * 2
            hl0 = load_lists(g0, 0)

            @pl.when(g0 + 1 < ng)
            def _():
                hl1 = load_lists(g0 + 1, 1)
                if nrb == 2:
                    hg0 = gather_rows(0, hl0)
                    hg1 = gather_rows(1, hl1)
                    rmw_group(0, hg0)
                    rmw_group(1, hg1)
                else:
                    rmw_group(0, gather_rows(0, hl0))
                    rmw_group(1, gather_rows(1, hl1))

            @pl.when(g0 + 1 >= ng)
            def _():
                rmw_group(0, gather_rows(0, hl0))
            return 0

        lax.fori_loop(0, (ng + 1) // 2, pair, 0)

        pltpu.sync_copy(acc.at[pl.ds(0, RANGE * d)],
                        out_hbm.at[c, pl.ds(lo * d, RANGE * d)])

    return seg_max


_segment_max_64 = _make_segment_max(64)
_segment_max_32 = _make_segment_max(32)


_BE = 2048


def _mlp_body(ea_ref, xg_ref, w1, b1, w2, b2, w3, b3, w4, b4, w5, b5,
              out_ref):
    f32 = jnp.float32
    h = jnp.maximum(jnp.dot(ea_ref[...], w1[...],
                            preferred_element_type=f32) + b1[...], 0.0)
    h = jnp.maximum(jnp.dot(h, w2[...],
                            preferred_element_type=f32) + b2[...], 0.0)
    h = jnp.maximum(jnp.dot(h, w3[...],
                            preferred_element_type=f32) + b3[...], 0.0)
    h = jnp.maximum(jnp.dot(h, w4[...],
                            preferred_element_type=f32) + b4[...], 0.0)
    w = jax.nn.sigmoid(jnp.dot(h, w5[...],
                               preferred_element_type=f32) + b5[...])
    acc = xg_ref[:, 0:1] * w[:, 0:128]
    for i in range(1, 7):
        acc = acc + xg_ref[:, i:i + 1] * w[:, i * 128:(i + 1) * 128]
    out_ref[...] = acc


def _mlp_msg(ea_pad, xg, p):
    ws = []
    for i in range(1, 6):
        ws.append(p[f"mlp_W{i}"])
        ws.append(p[f"mlp_b{i}"].reshape(1, -1))
    full = lambda a: pl.BlockSpec(a.shape, lambda i: (0,) * a.ndim)
    return pl.pallas_call(
        _mlp_body,
        grid=(E_PAD // _BE,),
        in_specs=[
            pl.BlockSpec((_BE, 3), lambda i: (i, 0)),
            pl.BlockSpec((_BE, 16), lambda i: (i, 0)),
        ] + [full(a) for a in ws],
        out_specs=pl.BlockSpec((_BE, 128), lambda i: (i, 0)),
        out_shape=jax.ShapeDtypeStruct((E_PAD, 128), jnp.float32),
    )(ea_pad, xg, *ws)


_BN = 2048


def _combine_mean_ab(aggp, degp, bias, wts, wpms, cb):
    d = wts.shape[1]

    def body(aggp_ref, degp_ref, bias_ref, wts_ref, wpms_ref, cb_ref,
             a_ref, b_ref, degc_ref):
        agg = aggp_ref[0] + aggp_ref[1]
        deg = degp_ref[0, :, 0:1] + degp_ref[1, :, 0:1]
        h = agg / jnp.maximum(deg, 1.0) + bias_ref[...]
        a_ref[...] = jnp.dot(h, wts_ref[...], precision=jax.lax.Precision.HIGHEST,
                             preferred_element_type=jnp.float32)
        b_ref[...] = jnp.dot(h, wpms_ref[...], precision=jax.lax.Precision.HIGHEST,
                             preferred_element_type=jnp.float32) + cb_ref[...]
        degc_ref[...] = jnp.broadcast_to(deg, (_BN, 16))

    full = lambda a: pl.BlockSpec(a.shape, lambda i: (0,) * a.ndim)
    return pl.pallas_call(
        body,
        grid=(N_PAD // _BN,),
        in_specs=[
            pl.BlockSpec((2, _BN, 128), lambda i: (0, i, 0)),
            pl.BlockSpec((2, _BN, 16), lambda i: (0, i, 0)),
            full(bias), full(wts), full(wpms), full(cb),
        ],
        out_specs=[
            pl.BlockSpec((_BN, d), lambda i: (i, 0)),
            pl.BlockSpec((_BN, d), lambda i: (i, 0)),
            pl.BlockSpec((_BN, 16), lambda i: (i, 0)),
        ],
        out_shape=[
            jax.ShapeDtypeStruct((N_PAD, d), jnp.float32),
            jax.ShapeDtypeStruct((N_PAD, d), jnp.float32),
            jax.ShapeDtypeStruct((N_PAD, 16), jnp.float32),
        ],
    )(aggp, degp, bias, wts, wpms, cb)


def _next_layer_ab(mp, bprev, degc, wts, wpms, cb):
    dp = bprev.shape[1]
    d = wts.shape[1]

    def body(mp_ref, bprev_ref, degc_ref, wts_ref, wpms_ref, cb_ref,
             a_ref, b_ref):
        m = jnp.maximum(mp_ref[0], mp_ref[1])
        h = jnp.where(degc_ref[:, 0:1] > 0.5, bprev_ref[...] + m, 0.0)
        a_ref[...] = jnp.dot(h, wts_ref[...], precision=jax.lax.Precision.HIGHEST,
                             preferred_element_type=jnp.float32)
        b_ref[...] = jnp.dot(h, wpms_ref[...], precision=jax.lax.Precision.HIGHEST,
                             preferred_element_type=jnp.float32) + cb_ref[...]

    full = lambda a: pl.BlockSpec(a.shape, lambda i: (0,) * a.ndim)
    return pl.pallas_call(
        body,
        grid=(N_PAD // _BN,),
        in_specs=[
            pl.BlockSpec((2, _BN, dp), lambda i: (0, i, 0)),
            pl.BlockSpec((_BN, dp), lambda i: (i, 0)),
            pl.BlockSpec((_BN, 16), lambda i: (i, 0)),
            full(wts), full(wpms), full(cb),
        ],
        out_specs=[
            pl.BlockSpec((_BN, d), lambda i: (i, 0)),
            pl.BlockSpec((_BN, d), lambda i: (i, 0)),
        ],
        out_shape=[
            jax.ShapeDtypeStruct((N_PAD, d), jnp.float32),
            jax.ShapeDtypeStruct((N_PAD, d), jnp.float32),
        ],
    )(mp, bprev, degc, wts, wpms, cb)


def _final_h(mp, bprev, degc):
    dp = bprev.shape[1]

    def body(mp_ref, bprev_ref, degc_ref, h_ref):
        m = jnp.maximum(mp_ref[0], mp_ref[1])
        h_ref[...] = jnp.where(degc_ref[:, 0:1] > 0.5, bprev_ref[...] + m,
                               0.0)

    return pl.pallas_call(
        body,
        grid=(N_PAD // _BN,),
        in_specs=[
            pl.BlockSpec((2, _BN, dp), lambda i: (0, i, 0)),
            pl.BlockSpec((_BN, dp), lambda i: (i, 0)),
            pl.BlockSpec((_BN, 16), lambda i: (i, 0)),
        ],
        out_specs=pl.BlockSpec((_BN, dp), lambda i: (i, 0)),
        out_shape=jax.ShapeDtypeStruct((N_PAD, dp), jnp.float32),
    )(mp, bprev, degc)


def kernel(x, edge_index, edge_attr, params):
    p = params
    src = edge_index[0].astype(jnp.int32)
    dst = edge_index[1].astype(jnp.int32)
    src_pad = jnp.pad(src, (0, E_PAD - E))
    dst_pad = jnp.pad(dst, (0, E_PAD - E), constant_values=DUMP_DST)
    ea_pad = jnp.pad(edge_attr, ((0, E_PAD - E), (0, 0)))
    xpad = jnp.pad(x, ((0, 0), (0, 16 - x.shape[1])))

    wts, wpms, cbs = [], [], []
    for i in (1, 2, 3):
        scale = p[f"ec{i}_g"] / jnp.sqrt(p[f"ec{i}_rv"] + 1e-5)
        shift = p[f"ec{i}_b"] - p[f"ec{i}_rm"] * scale
        wts.append(p[f"ec{i}_Wt"] * scale[None, :])
        wpms.append((p[f"ec{i}_Wp"] - p[f"ec{i}_Wt"]) * scale[None, :])
        cbs.append(((p[f"ec{i}_bt"] + p[f"ec{i}_bp"]) * scale
                    + shift).reshape(1, -1))

    xg = _gather_x(src_pad, xpad)
    slist, dlist, cnt = _bucketize(dst_pad, src_pad)
    msg = _mlp_msg(ea_pad, xg, p)
    aggp, degp = _segment_sum(dst_pad, msg)
    a1, b1, degc = _combine_mean_ab(
        aggp, degp, p["nnconv_bias"].reshape(1, -1), wts[0], wpms[0], cbs[0])
    mp1 = _segment_max_64(slist, dlist, cnt, a1).reshape(2, N_PAD, 64)
    a2, b2 = _next_layer_ab(mp1, b1, degc, wts[1], wpms[1], cbs[1])
    mp2 = _segment_max_32(slist, dlist, cnt, a2).reshape(2, N_PAD, 32)
    a3, b3 = _next_layer_ab(mp2, b2, degc, wts[2], wpms[2], cbs[2])
    mp3 = _segment_max_32(slist, dlist, cnt, a3).reshape(2, N_PAD, 32)
    h = _final_h(mp3, b3, degc)
    return h[:N]


# speedup vs baseline: 1.0673x; 1.0673x over previous
import functools

import jax
import jax.numpy as jnp
from jax import lax
from jax.experimental import pallas as pl
from jax.experimental.pallas import tpu as pltpu
from jax.experimental.pallas import tpu_sc as plsc

N = 10000
E = 160000
N_PAD = 10240
RANGE = 640
TILE_E = E // 32
HALF_E = E // 2

_mesh = plsc.VectorSubcoreMesh(core_axis_name="c", subcore_axis_name="s")


@functools.partial(
    pl.kernel,
    out_type=jax.ShapeDtypeStruct((E, 16), jnp.float32),
    mesh=_mesh,
    compiler_params=pltpu.CompilerParams(use_tc_tiling_on_sc=False,
                                         needs_layout_passes=False),
    scratch_types=[
        pltpu.VMEM((2, 512), jnp.int32),
        pltpu.VMEM((2, 512, 16), jnp.float32),
        pltpu.SemaphoreType.DMA,
        pltpu.SemaphoreType.DMA,
    ],
)
def _gather_x(src_hbm, xpad_hbm, out_hbm, idx_v, rows_v, s0, s1):
    wid = lax.axis_index("s") * 2 + lax.axis_index("c")
    base = wid * TILE_E
    sems = (s0, s1)

    def pair(b0, n0, b1, n1):
        pltpu.sync_copy(src_hbm.at[pl.ds(b0, n0)], idx_v.at[0, pl.ds(0, n0)])
        h0 = pltpu.async_copy(xpad_hbm.at[idx_v.at[0, pl.ds(0, n0)]],
                              rows_v.at[0, pl.ds(0, n0)], sems[0])
        pltpu.sync_copy(src_hbm.at[pl.ds(b1, n1)], idx_v.at[1, pl.ds(0, n1)])
        h1 = pltpu.async_copy(xpad_hbm.at[idx_v.at[1, pl.ds(0, n1)]],
                              rows_v.at[1, pl.ds(0, n1)], sems[1])
        h0.wait()
        pltpu.sync_copy(rows_v.at[0, pl.ds(0, n0)],
                        out_hbm.at[pl.ds(b0, n0)])
        h1.wait()
        pltpu.sync_copy(rows_v.at[1, pl.ds(0, n1)],
                        out_hbm.at[pl.ds(b1, n1)])

    def body(j, _):
        b = base + j * 1024
        pair(b, 512, b + 512, 512)
        return 0

    lax.fori_loop(0, 4, body, 0)
    pair(base + 4096, 512, base + 4608, 392)


@functools.partial(
    pl.kernel,
    out_type=(
        jax.ShapeDtypeStruct((2, N_PAD, 128), jnp.float32),
        jax.ShapeDtypeStruct((2, N_PAD, 16), jnp.float32),
    ),
    mesh=_mesh,
    compiler_params=pltpu.CompilerParams(use_tc_tiling_on_sc=False,
                                         needs_layout_passes=False),
    scratch_types=[
        pltpu.VMEM((2, 128), jnp.int32),
        pltpu.VMEM((2, 128, 128), jnp.float32),
        pltpu.VMEM((128, 16), jnp.float32),
        pltpu.VMEM((128, 16), jnp.float32),
        pltpu.VMEM_SHARED((N_PAD + 128, 128), jnp.float32),
        pltpu.VMEM_SHARED((N_PAD + 128, 16), jnp.float32),
    ] + [pltpu.SemaphoreType.DMA] * 8,
)
def _segment_sum(dst_hbm, msg_hbm, aggp_hbm, degp_hbm,
                 idx_v, rows_v, ones_v, zd_v, agg_sh, deg_sh,
                 li0, li1, lm0, lm1, sa0, sa1, sd0, sd1):
    c = lax.axis_index("c")
    s = lax.axis_index("s")
    zero16 = jnp.zeros((16,), jnp.float32)
    one16 = jnp.ones((16,), jnp.float32)
    li = (li0, li1)
    lm = (lm0, lm1)
    sa = (sa0, sa1)
    sd = (sd0, sd1)

    def initrow(r, _):
        for j in range(8):
            rows_v[0, r, pl.ds(j * 16, 16)] = zero16
        ones_v[r, pl.ds(0, 16)] = one16
        zd_v[r, pl.ds(0, 16)] = zero16
        return 0

    lax.fori_loop(0, 128, initrow, 0)

    for k in range(RANGE // 128):
        pltpu.sync_copy(rows_v.at[0],
                        agg_sh.at[pl.ds(s * RANGE + k * 128, 128)])
        pltpu.sync_copy(zd_v, deg_sh.at[pl.ds(s * RANGE + k * 128, 128)])

    @pl.when(s == 0)
    def _():
        pltpu.sync_copy(rows_v.at[0], agg_sh.at[pl.ds(N_PAD, 128)])
        pltpu.sync_copy(zd_v, deg_sh.at[pl.ds(N_PAD, 128)])

    plsc.subcore_barrier()

    base = (s * 2 + c) * TILE_E

    def do_pair(b0, n0, b1, n1):
        hl = []
        for q, (b, n) in enumerate(((b0, n0), (b1, n1))):
            hl.append((pltpu.async_copy(dst_hbm.at[pl.ds(b, n)],
                                        idx_v.at[q, pl.ds(0, n)], li[q]),
                       pltpu.async_copy(msg_hbm.at[pl.ds(b, n)],
                                        rows_v.at[q, pl.ds(0, n)], lm[q])))
        hs = []
        for q in range(2):
            hl[q][0].wait()
            hl[q][1].wait()
            hs.append((pltpu.async_copy(rows_v.at[q],
                                        agg_sh.at[idx_v.at[q]],
                                        sa[q], add=True),
                       pltpu.async_copy(ones_v,
                                        deg_sh.at[idx_v.at[q]],
                                        sd[q], add=True)))
        for q in range(2):
            hs[q][0].wait()
            hs[q][1].wait()

    def body(j2, _):
        b = base + j2 * 256
        do_pair(b, 128, b + 128, 128)
        return 0

    lax.fori_loop(0, 19, body, 0)
    dump16 = jnp.full((16,), N_PAD, jnp.int32)
    for t in range(8):
        idx_v[1, pl.ds(t * 16, 16)] = dump16
    do_pair(base + 4864, 128, base + 4992, 8)
    plsc.subcore_barrier()

    pltpu.sync_copy(agg_sh.at[pl.ds(s * RANGE, RANGE)],
                    aggp_hbm.at[c, pl.ds(s * RANGE, RANGE)])
    pltpu.sync_copy(deg_sh.at[pl.ds(s * RANGE, RANGE)],
                    degp_hbm.at[c, pl.ds(s * RANGE, RANGE)])


CAP = HALF_E + 512
CAPT = 32768
LBUF = 36864


@functools.partial(
    pl.kernel,
    out_type=(
        jax.ShapeDtypeStruct((32, CAP), jnp.int32),
        jax.ShapeDtypeStruct((32, CAP), jnp.int32),
        jax.ShapeDtypeStruct((32, 16), jnp.int32),
    ),
    mesh=_mesh,
    compiler_params=pltpu.CompilerParams(use_tc_tiling_on_sc=False,
                                         needs_layout_passes=False),
    scratch_types=[
        pltpu.VMEM((2, 2048), jnp.int32),
        pltpu.VMEM((2, 2048), jnp.int32),
        pltpu.VMEM((LBUF,), jnp.int32),
        pltpu.VMEM((LBUF,), jnp.int32),
        pltpu.VMEM((16,), jnp.int32),
    ] + [pltpu.SemaphoreType.DMA] * 4,
)
def _bucketize(dst_hbm, src_hbm, slist_hbm, dlist_hbm, cnt_hbm,
               dv, sv, sl, dl, cb, c0, c1, c2, c3):
    c = lax.axis_index("c")
    s = lax.axis_index("s")
    wid = s * 2 + c
    lo = s * RANGE
    zero16i = jnp.zeros((16,), jnp.int32)
    sent16 = jnp.full((16,), RANGE, jnp.int32)
    ebase = c * HALF_E

    def flush(nblk, woff, offbase):
        def cp(i, _):
            so = pl.multiple_of(offbase + i * 128, 128)
            ho = pl.multiple_of(woff + i * 128, 128)
            pltpu.sync_copy(sl.at[pl.ds(so, 128)],
                            slist_hbm.at[wid, pl.ds(ho, 128)])
            pltpu.sync_copy(dl.at[pl.ds(so, 128)],
                            dlist_hbm.at[wid, pl.ds(ho, 128)])
            return 0
        lax.fori_loop(0, nblk, cp, 0)

    csem = ((c0, c1), (c2, c3))
    CHUNK = 2000

    def load_chunk(ch, p):
        cb2 = ebase + ch * CHUNK
        pltpu.async_copy(dst_hbm.at[pl.ds(cb2, CHUNK)],
                         dv.at[p, pl.ds(0, CHUNK)], csem[p][0])
        pltpu.async_copy(src_hbm.at[pl.ds(cb2, CHUNK)],
                         sv.at[p, pl.ds(0, CHUNK)], csem[p][1])

    def wait_chunk(p):
        pltpu.make_async_copy(dst_hbm.at[pl.ds(0, CHUNK)],
                              dv.at[p, pl.ds(0, CHUNK)], csem[p][0]).wait()
        pltpu.make_async_copy(src_hbm.at[pl.ds(0, CHUNK)],
                              sv.at[p, pl.ds(0, CHUNK)], csem[p][1]).wait()

    def scan_chunk(p, carry):
        off, woff = carry
        wait_chunk(p)

        def scan(v, o):
            dvec = dv[p, pl.ds(v * 16, 16)]
            svec = sv[p, pl.ds(v * 16, 16)]
            m = (dvec >= lo) & (dvec < lo + RANGE)
            pos = o + plsc.cumsum(m.astype(jnp.int32)) - 1
            plsc.store_scatter(sl, [pos], svec, mask=m)
            plsc.store_scatter(dl, [pos], dvec - lo, mask=m)
            return pos[15] + 1

        off = lax.fori_loop(0, 125, scan, off)

        nblk = jnp.where(off >= CAPT, off // 128, 0)
        flush(nblk, woff, 0)
        rem_base = nblk * 128
        for t in range(8):
            sl[pl.ds(t * 16, 16)] = sl[pl.ds(rem_base + t * 16, 16)]
            dl[pl.ds(t * 16, 16)] = dl[pl.ds(rem_base + t * 16, 16)]
        return off - rem_base, woff + rem_base

    npairs = HALF_E // CHUNK // 2
    load_chunk(0, 0)

    def pair_body(kk, carry):
        ch0 = kk * 2
        load_chunk(ch0 + 1, 1)
        carry = scan_chunk(0, carry)

        @pl.when(kk < npairs - 1)
        def _():
            load_chunk(ch0 + 2, 0)
        return scan_chunk(1, carry)

    off, woff = lax.fori_loop(0, npairs, pair_body, (0, 0))

    for t in range(32):
        sl[pl.ds(off + t * 16, 16)] = zero16i
        dl[pl.ds(off + t * 16, 16)] = sent16
    total = woff + off
    total_p = ((total + 511) // 512) * 512
    off_p = total_p - woff
    flush(off_p // 128, woff, 0)
    cb[pl.ds(0, 16)] = jnp.full((16,), total_p, jnp.int32)
    pltpu.sync_copy(cb, cnt_hbm.at[wid])


def _make_segment_max(d):
    nvr = d // 16

    @functools.partial(
        pl.kernel,
        out_type=jax.ShapeDtypeStruct((2, N_PAD * d), jnp.float32),
        mesh=_mesh,
        compiler_params=pltpu.CompilerParams(use_tc_tiling_on_sc=False,
                                         needs_layout_passes=False),
        scratch_types=[
            pltpu.VMEM((2, 512), jnp.int32),
            pltpu.VMEM((2, 512), jnp.int32),
            pltpu.VMEM((16,), jnp.int32),
            pltpu.VMEM((2 if d <= 32 else 1, 4, 128, d), jnp.float32),
            pltpu.VMEM(((RANGE + 8) * d,), jnp.float32),
        ] + [pltpu.SemaphoreType.DMA] * 12,
    )
    def seg_max(slist_hbm, dlist_hbm, cnt_hbm, a_hbm, out_hbm,
                sbuf, dbuf, cb, rows_v, acc,
                s0, s1, s2, s3, s4, s5, s6, s7, l0, l1, l2, l3):
        c = lax.axis_index("c")
        s = lax.axis_index("s")
        wid = s * 2 + c
        lo = s * RANGE
        neg = jnp.full((16,), -3.0e38, jnp.float32)
        sems = ((s0, s1, s2, s3), (s4, s5, s6, s7))
        lsem = ((l0, l1), (l2, l3))

        def initrow(r, _):
            acc[pl.ds(r * 16, 16)] = neg
            return 0

        lax.fori_loop(0, (RANGE + 8) * d // 16, initrow, 0)

        pltpu.sync_copy(cnt_hbm.at[wid], cb)
        total = cb[pl.ds(0, 16)][0]
        ng = total // 512

        def load_lists(g, p):
            gb = g * 512
            return (pltpu.async_copy(slist_hbm.at[wid, pl.ds(gb, 512)],
                                     sbuf.at[p], lsem[p][0]),
                    pltpu.async_copy(dlist_hbm.at[wid, pl.ds(gb, 512)],
                                     dbuf.at[p], lsem[p][1]))

        nrb = 2 if d <= 32 else 1

        def gather_rows(p, hl):
            hl[0].wait()
            hl[1].wait()
            return [pltpu.async_copy(
                        a_hbm.at[sbuf.at[p, pl.ds(q * 128, 128)]],
                        rows_v.at[p % nrb, q], sems[p][q])
                    for q in range(4)]

        def rmw_group(p, hg):
            for q in range(4):
                hg[q].wait()

                def rmw(g8, _):
                    basev = dbuf[p, pl.ds(q * 128 + g8 * 16, 16)] * d
                    for b in range(16):
                        ab = basev[b]
                        for j in range(nvr):
                            cur = acc[pl.ds(ab + j * 16, 16)]
                            acc[pl.ds(ab + j * 16, 16)] = jnp.maximum(
                                cur,
                                rows_v[p % nrb, q,
                                       g8 * 16 + b, pl.ds(j * 16, 16)])
                    return 0

                lax.fori_loop(0, 8, rmw, 0)

        def pair(kk, _):
            g0 = kk * 2
            hl0 = load_lists(g0, 0)

            @pl.when(g0 + 1 < ng)
            def _():
                hl1 = load_lists(g0 + 1, 1)
                if nrb == 2:
                    hg0 = gather_rows(0, hl0)
                    hg1 = gather_rows(1, hl1)
                    rmw_group(0, hg0)
                    rmw_group(1, hg1)
                else:
                    rmw_group(0, gather_rows(0, hl0))
                    rmw_group(1, gather_rows(1, hl1))

            @pl.when(g0 + 1 >= ng)
            def _():
                rmw_group(0, gather_rows(0, hl0))
            return 0

        lax.fori_loop(0, (ng + 1) // 2, pair, 0)

        pltpu.sync_copy(acc.at[pl.ds(0, RANGE * d)],
                        out_hbm.at[c, pl.ds(lo * d, RANGE * d)])

    return seg_max


_segment_max_64 = _make_segment_max(64)
_segment_max_32 = _make_segment_max(32)


_BE = 1280


def _mlp_body(ea_ref, xg_ref, w1, b1, w2, b2, w3, b3, w4, b4, w5, b5,
              out_ref):
    f32 = jnp.float32
    h = jnp.maximum(jnp.dot(ea_ref[...], w1[...],
                            preferred_element_type=f32) + b1[...], 0.0)
    h = jnp.maximum(jnp.dot(h, w2[...],
                            preferred_element_type=f32) + b2[...], 0.0)
    h = jnp.maximum(jnp.dot(h, w3[...],
                            preferred_element_type=f32) + b3[...], 0.0)
    h = jnp.maximum(jnp.dot(h, w4[...],
                            preferred_element_type=f32) + b4[...], 0.0)
    w = jax.nn.sigmoid(jnp.dot(h, w5[...],
                               preferred_element_type=f32) + b5[...])
    acc = xg_ref[:, 0:1] * w[:, 0:128]
    for i in range(1, 7):
        acc = acc + xg_ref[:, i:i + 1] * w[:, i * 128:(i + 1) * 128]
    out_ref[...] = acc


def _mlp_msg(ea, xg, p):
    ws = []
    for i in range(1, 6):
        ws.append(p[f"mlp_W{i}"])
        ws.append(p[f"mlp_b{i}"].reshape(1, -1))
    full = lambda a: pl.BlockSpec(a.shape, lambda i: (0,) * a.ndim)
    return pl.pallas_call(
        _mlp_body,
        grid=(E // _BE,),
        in_specs=[
            pl.BlockSpec((_BE, 3), lambda i: (i, 0)),
            pl.BlockSpec((_BE, 16), lambda i: (i, 0)),
        ] + [full(a) for a in ws],
        out_specs=pl.BlockSpec((_BE, 128), lambda i: (i, 0)),
        out_shape=jax.ShapeDtypeStruct((E, 128), jnp.float32),
    )(ea, xg, *ws)


_BN = 2048


def _combine_mean_ab(aggp, degp, bias, wts, wpms, cb):
    d = wts.shape[1]

    def body(aggp_ref, degp_ref, bias_ref, wts_ref, wpms_ref, cb_ref,
             a_ref, b_ref, degc_ref):
        agg = aggp_ref[0] + aggp_ref[1]
        deg = degp_ref[0, :, 0:1] + degp_ref[1, :, 0:1]
        h = agg / jnp.maximum(deg, 1.0) + bias_ref[...]
        a_ref[...] = jnp.dot(h, wts_ref[...], precision=jax.lax.Precision.HIGHEST,
                             preferred_element_type=jnp.float32)
        b_ref[...] = jnp.dot(h, wpms_ref[...], precision=jax.lax.Precision.HIGHEST,
                             preferred_element_type=jnp.float32) + cb_ref[...]
        degc_ref[...] = jnp.broadcast_to(deg, (_BN, 16))

    full = lambda a: pl.BlockSpec(a.shape, lambda i: (0,) * a.ndim)
    return pl.pallas_call(
        body,
        grid=(N_PAD // _BN,),
        in_specs=[
            pl.BlockSpec((2, _BN, 128), lambda i: (0, i, 0)),
            pl.BlockSpec((2, _BN, 16), lambda i: (0, i, 0)),
            full(bias), full(wts), full(wpms), full(cb),
        ],
        out_specs=[
            pl.BlockSpec((_BN, d), lambda i: (i, 0)),
            pl.BlockSpec((_BN, d), lambda i: (i, 0)),
            pl.BlockSpec((_BN, 16), lambda i: (i, 0)),
        ],
        out_shape=[
            jax.ShapeDtypeStruct((N_PAD, d), jnp.float32),
            jax.ShapeDtypeStruct((N_PAD, d), jnp.float32),
            jax.ShapeDtypeStruct((N_PAD, 16), jnp.float32),
        ],
    )(aggp, degp, bias, wts, wpms, cb)


def _next_layer_ab(mp, bprev, degc, wts, wpms, cb):
    dp = bprev.shape[1]
    d = wts.shape[1]

    def body(mp_ref, bprev_ref, degc_ref, wts_ref, wpms_ref, cb_ref,
             a_ref, b_ref):
        m = jnp.maximum(mp_ref[0], mp_ref[1])
        h = jnp.where(degc_ref[:, 0:1] > 0.5, bprev_ref[...] + m, 0.0)
        a_ref[...] = jnp.dot(h, wts_ref[...], precision=jax.lax.Precision.HIGHEST,
                             preferred_element_type=jnp.float32)
        b_ref[...] = jnp.dot(h, wpms_ref[...], precision=jax.lax.Precision.HIGHEST,
                             preferred_element_type=jnp.float32) + cb_ref[...]

    full = lambda a: pl.BlockSpec(a.shape, lambda i: (0,) * a.ndim)
    return pl.pallas_call(
        body,
        grid=(N_PAD // _BN,),
        in_specs=[
            pl.BlockSpec((2, _BN, dp), lambda i: (0, i, 0)),
            pl.BlockSpec((_BN, dp), lambda i: (i, 0)),
            pl.BlockSpec((_BN, 16), lambda i: (i, 0)),
            full(wts), full(wpms), full(cb),
        ],
        out_specs=[
            pl.BlockSpec((_BN, d), lambda i: (i, 0)),
            pl.BlockSpec((_BN, d), lambda i: (i, 0)),
        ],
        out_shape=[
            jax.ShapeDtypeStruct((N_PAD, d), jnp.float32),
            jax.ShapeDtypeStruct((N_PAD, d), jnp.float32),
        ],
    )(mp, bprev, degc, wts, wpms, cb)


def _final_h(mp, bprev, degc):
    dp = bprev.shape[1]

    def body(mp_ref, bprev_ref, degc_ref, h_ref):
        m = jnp.maximum(mp_ref[0], mp_ref[1])
        h_ref[...] = jnp.where(degc_ref[:, 0:1] > 0.5, bprev_ref[...] + m,
                               0.0)

    return pl.pallas_call(
        body,
        grid=(N_PAD // _BN,),
        in_specs=[
            pl.BlockSpec((2, _BN, dp), lambda i: (0, i, 0)),
            pl.BlockSpec((_BN, dp), lambda i: (i, 0)),
            pl.BlockSpec((_BN, 16), lambda i: (i, 0)),
        ],
        out_specs=pl.BlockSpec((_BN, dp), lambda i: (i, 0)),
        out_shape=jax.ShapeDtypeStruct((N_PAD, dp), jnp.float32),
    )(mp, bprev, degc)


def kernel(x, edge_index, edge_attr, params):
    p = params
    src = edge_index[0].astype(jnp.int32)
    dst = edge_index[1].astype(jnp.int32)
    xpad = jnp.pad(x, ((0, 0), (0, 16 - x.shape[1])))

    wts, wpms, cbs = [], [], []
    for i in (1, 2, 3):
        scale = p[f"ec{i}_g"] / jnp.sqrt(p[f"ec{i}_rv"] + 1e-5)
        shift = p[f"ec{i}_b"] - p[f"ec{i}_rm"] * scale
        wts.append(p[f"ec{i}_Wt"] * scale[None, :])
        wpms.append((p[f"ec{i}_Wp"] - p[f"ec{i}_Wt"]) * scale[None, :])
        cbs.append(((p[f"ec{i}_bt"] + p[f"ec{i}_bp"]) * scale
                    + shift).reshape(1, -1))

    xg = _gather_x(src, xpad)
    slist, dlist, cnt = _bucketize(dst, src)
    msg = _mlp_msg(edge_attr, xg, p)
    aggp, degp = _segment_sum(dst, msg)
    a1, b1, degc = _combine_mean_ab(
        aggp, degp, p["nnconv_bias"].reshape(1, -1), wts[0], wpms[0], cbs[0])
    mp1 = _segment_max_64(slist, dlist, cnt, a1).reshape(2, N_PAD, 64)
    a2, b2 = _next_layer_ab(mp1, b1, degc, wts[1], wpms[1], cbs[1])
    mp2 = _segment_max_32(slist, dlist, cnt, a2).reshape(2, N_PAD, 32)
    a3, b3 = _next_layer_ab(mp2, b2, degc, wts[2], wpms[2], cbs[2])
    mp3 = _segment_max_32(slist, dlist, cnt, a3).reshape(2, N_PAD, 32)
    h = _final_h(mp3, b3, degc)
    return h[:N]


# speedup vs baseline: 1.0808x; 1.0127x over previous
import functools

import jax
import jax.numpy as jnp
from jax import lax
from jax.experimental import pallas as pl
from jax.experimental.pallas import tpu as pltpu
from jax.experimental.pallas import tpu_sc as plsc

N = 10000
E = 160000
N_PAD = 10240
RANGE = 640
TILE_E = E // 32
HALF_E = E // 2

_mesh = plsc.VectorSubcoreMesh(core_axis_name="c", subcore_axis_name="s")


@functools.partial(
    pl.kernel,
    out_type=jax.ShapeDtypeStruct((E, 16), jnp.float32),
    mesh=_mesh,
    compiler_params=pltpu.CompilerParams(use_tc_tiling_on_sc=False,
                                         needs_layout_passes=False),
    scratch_types=[
        pltpu.VMEM((2, 512), jnp.int32),
        pltpu.VMEM((2, 512, 16), jnp.float32),
        pltpu.SemaphoreType.DMA,
        pltpu.SemaphoreType.DMA,
    ],
)
def _gather_x(src_hbm, xpad_hbm, out_hbm, idx_v, rows_v, s0, s1):
    wid = lax.axis_index("s") * 2 + lax.axis_index("c")
    base = wid * TILE_E
    sems = (s0, s1)

    def pair(b0, n0, b1, n1):
        pltpu.sync_copy(src_hbm.at[pl.ds(b0, n0)], idx_v.at[0, pl.ds(0, n0)])
        h0 = pltpu.async_copy(xpad_hbm.at[idx_v.at[0, pl.ds(0, n0)]],
                              rows_v.at[0, pl.ds(0, n0)], sems[0])
        pltpu.sync_copy(src_hbm.at[pl.ds(b1, n1)], idx_v.at[1, pl.ds(0, n1)])
        h1 = pltpu.async_copy(xpad_hbm.at[idx_v.at[1, pl.ds(0, n1)]],
                              rows_v.at[1, pl.ds(0, n1)], sems[1])
        h0.wait()
        pltpu.sync_copy(rows_v.at[0, pl.ds(0, n0)],
                        out_hbm.at[pl.ds(b0, n0)])
        h1.wait()
        pltpu.sync_copy(rows_v.at[1, pl.ds(0, n1)],
                        out_hbm.at[pl.ds(b1, n1)])

    def body(j, _):
        b = base + j * 1024
        pair(b, 512, b + 512, 512)
        return 0

    lax.fori_loop(0, 4, body, 0)
    pair(base + 4096, 512, base + 4608, 392)


@functools.partial(
    pl.kernel,
    out_type=(
        jax.ShapeDtypeStruct((2, N_PAD, 128), jnp.float32),
        jax.ShapeDtypeStruct((2, N_PAD, 16), jnp.float32),
    ),
    mesh=_mesh,
    compiler_params=pltpu.CompilerParams(use_tc_tiling_on_sc=False,
                                         needs_layout_passes=False),
    scratch_types=[
        pltpu.VMEM((2, 128), jnp.int32),
        pltpu.VMEM((2, 128, 128), jnp.float32),
        pltpu.VMEM((128, 16), jnp.float32),
        pltpu.VMEM((128, 16), jnp.float32),
        pltpu.VMEM_SHARED((N_PAD + 128, 128), jnp.float32),
        pltpu.VMEM_SHARED((N_PAD + 128, 16), jnp.float32),
    ] + [pltpu.SemaphoreType.DMA] * 8,
)
def _segment_sum(dst_hbm, msg_hbm, aggp_hbm, degp_hbm,
                 idx_v, rows_v, ones_v, zd_v, agg_sh, deg_sh,
                 li0, li1, lm0, lm1, sa0, sa1, sd0, sd1):
    c = lax.axis_index("c")
    s = lax.axis_index("s")
    zero16 = jnp.zeros((16,), jnp.float32)
    one16 = jnp.ones((16,), jnp.float32)
    li = (li0, li1)
    lm = (lm0, lm1)
    sa = (sa0, sa1)
    sd = (sd0, sd1)

    def initrow(r, _):
        for j in range(8):
            rows_v[0, r, pl.ds(j * 16, 16)] = zero16
        ones_v[r, pl.ds(0, 16)] = one16
        zd_v[r, pl.ds(0, 16)] = zero16
        return 0

    lax.fori_loop(0, 128, initrow, 0)

    for k in range(RANGE // 128):
        pltpu.sync_copy(rows_v.at[0],
                        agg_sh.at[pl.ds(s * RANGE + k * 128, 128)])
        pltpu.sync_copy(zd_v, deg_sh.at[pl.ds(s * RANGE + k * 128, 128)])

    @pl.when(s == 0)
    def _():
        pltpu.sync_copy(rows_v.at[0], agg_sh.at[pl.ds(N_PAD, 128)])
        pltpu.sync_copy(zd_v, deg_sh.at[pl.ds(N_PAD, 128)])

    plsc.subcore_barrier()

    base = (s * 2 + c) * TILE_E

    def do_pair(b0, n0, b1, n1):
        hl = []
        for q, (b, n) in enumerate(((b0, n0), (b1, n1))):
            hl.append((pltpu.async_copy(dst_hbm.at[pl.ds(b, n)],
                                        idx_v.at[q, pl.ds(0, n)], li[q]),
                       pltpu.async_copy(msg_hbm.at[pl.ds(b, n)],
                                        rows_v.at[q, pl.ds(0, n)], lm[q])))
        hs = []
        for q in range(2):
            hl[q][0].wait()
            hl[q][1].wait()
            hs.append((pltpu.async_copy(rows_v.at[q],
                                        agg_sh.at[idx_v.at[q]],
                                        sa[q], add=True),
                       pltpu.async_copy(ones_v,
                                        deg_sh.at[idx_v.at[q]],
                                        sd[q], add=True)))
        for q in range(2):
            hs[q][0].wait()
            hs[q][1].wait()

    def body(j2, _):
        b = base + j2 * 256
        do_pair(b, 128, b + 128, 128)
        return 0

    lax.fori_loop(0, 19, body, 0)
    dump16 = jnp.full((16,), N_PAD, jnp.int32)
    for t in range(8):
        idx_v[1, pl.ds(t * 16, 16)] = dump16
    do_pair(base + 4864, 128, base + 4992, 8)
    plsc.subcore_barrier()

    pltpu.sync_copy(agg_sh.at[pl.ds(s * RANGE, RANGE)],
                    aggp_hbm.at[c, pl.ds(s * RANGE, RANGE)])
    pltpu.sync_copy(deg_sh.at[pl.ds(s * RANGE, RANGE)],
                    degp_hbm.at[c, pl.ds(s * RANGE, RANGE)])


CAP = HALF_E + 512
CAPT = 32768
LBUF = 36864


@functools.partial(
    pl.kernel,
    out_type=(
        jax.ShapeDtypeStruct((32, CAP), jnp.int32),
        jax.ShapeDtypeStruct((32, CAP), jnp.int32),
        jax.ShapeDtypeStruct((32, 16), jnp.int32),
    ),
    mesh=_mesh,
    compiler_params=pltpu.CompilerParams(use_tc_tiling_on_sc=False,
                                         needs_layout_passes=False),
    scratch_types=[
        pltpu.VMEM((2, 2048), jnp.int32),
        pltpu.VMEM((2, 2048), jnp.int32),
        pltpu.VMEM((LBUF,), jnp.int32),
        pltpu.VMEM((LBUF,), jnp.int32),
        pltpu.VMEM((16,), jnp.int32),
    ] + [pltpu.SemaphoreType.DMA] * 4,
)
def _bucketize(dst_hbm, src_hbm, slist_hbm, dlist_hbm, cnt_hbm,
               dv, sv, sl, dl, cb, c0, c1, c2, c3):
    c = lax.axis_index("c")
    s = lax.axis_index("s")
    wid = s * 2 + c
    lo = s * RANGE
    zero16i = jnp.zeros((16,), jnp.int32)
    sent16 = jnp.full((16,), RANGE, jnp.int32)
    ebase = c * HALF_E

    def flush(nblk, woff, offbase):
        def cp(i, _):
            so = pl.multiple_of(offbase + i * 128, 128)
            ho = pl.multiple_of(woff + i * 128, 128)
            pltpu.sync_copy(sl.at[pl.ds(so, 128)],
                            slist_hbm.at[wid, pl.ds(ho, 128)])
            pltpu.sync_copy(dl.at[pl.ds(so, 128)],
                            dlist_hbm.at[wid, pl.ds(ho, 128)])
            return 0
        lax.fori_loop(0, nblk, cp, 0)

    csem = ((c0, c1), (c2, c3))
    CHUNK = 2000

    def load_chunk(ch, p):
        cb2 = ebase + ch * CHUNK
        pltpu.async_copy(dst_hbm.at[pl.ds(cb2, CHUNK)],
                         dv.at[p, pl.ds(0, CHUNK)], csem[p][0])
        pltpu.async_copy(src_hbm.at[pl.ds(cb2, CHUNK)],
                         sv.at[p, pl.ds(0, CHUNK)], csem[p][1])

    def wait_chunk(p):
        pltpu.make_async_copy(dst_hbm.at[pl.ds(0, CHUNK)],
                              dv.at[p, pl.ds(0, CHUNK)], csem[p][0]).wait()
        pltpu.make_async_copy(src_hbm.at[pl.ds(0, CHUNK)],
                              sv.at[p, pl.ds(0, CHUNK)], csem[p][1]).wait()

    def scan_chunk(p, carry):
        off, woff = carry
        wait_chunk(p)

        def scan(v, o):
            dvec = dv[p, pl.ds(v * 16, 16)]
            svec = sv[p, pl.ds(v * 16, 16)]
            m = (dvec >= lo) & (dvec < lo + RANGE)
            pos = o + plsc.cumsum(m.astype(jnp.int32)) - 1
            plsc.store_scatter(sl, [pos], svec, mask=m)
            plsc.store_scatter(dl, [pos], dvec - lo, mask=m)
            return pos[15] + 1

        off = lax.fori_loop(0, 125, scan, off)

        nblk = jnp.where(off >= CAPT, off // 128, 0)
        flush(nblk, woff, 0)
        rem_base = nblk * 128
        for t in range(8):
            sl[pl.ds(t * 16, 16)] = sl[pl.ds(rem_base + t * 16, 16)]
            dl[pl.ds(t * 16, 16)] = dl[pl.ds(rem_base + t * 16, 16)]
        return off - rem_base, woff + rem_base

    npairs = HALF_E // CHUNK // 2
    load_chunk(0, 0)

    def pair_body(kk, carry):
        ch0 = kk * 2
        load_chunk(ch0 + 1, 1)
        carry = scan_chunk(0, carry)

        @pl.when(kk < npairs - 1)
        def _():
            load_chunk(ch0 + 2, 0)
        return scan_chunk(1, carry)

    off, woff = lax.fori_loop(0, npairs, pair_body, (0, 0))

    for t in range(32):
        sl[pl.ds(off + t * 16, 16)] = zero16i
        dl[pl.ds(off + t * 16, 16)] = sent16
    total = woff + off
    total_p = ((total + 511) // 512) * 512
    off_p = total_p - woff
    flush(off_p // 128, woff, 0)
    cb[pl.ds(0, 16)] = jnp.full((16,), total_p, jnp.int32)
    pltpu.sync_copy(cb, cnt_hbm.at[wid])


def _make_segment_max(d):
    nvr = d // 16

    @functools.partial(
        pl.kernel,
        out_type=jax.ShapeDtypeStruct((2, N_PAD * d), jnp.float32),
        mesh=_mesh,
        compiler_params=pltpu.CompilerParams(use_tc_tiling_on_sc=False,
                                         needs_layout_passes=False),
        scratch_types=[
            pltpu.VMEM((2, 512), jnp.int32),
            pltpu.VMEM((2, 512), jnp.int32),
            pltpu.VMEM((16,), jnp.int32),
            pltpu.VMEM((2, 4, 128, d), jnp.float32),
            pltpu.VMEM(((RANGE + 8) * d,), jnp.float32),
        ] + [pltpu.SemaphoreType.DMA] * 12,
    )
    def seg_max(slist_hbm, dlist_hbm, cnt_hbm, a_hbm, out_hbm,
                sbuf, dbuf, cb, rows_v, acc,
                s0, s1, s2, s3, s4, s5, s6, s7, l0, l1, l2, l3):
        c = lax.axis_index("c")
        s = lax.axis_index("s")
        wid = s * 2 + c
        lo = s * RANGE
        neg = jnp.full((16,), -3.0e38, jnp.float32)
        sems = ((s0, s1, s2, s3), (s4, s5, s6, s7))
        lsem = ((l0, l1), (l2, l3))

        def initrow(r, _):
            acc[pl.ds(r * 16, 16)] = neg
            return 0

        lax.fori_loop(0, (RANGE + 8) * d // 16, initrow, 0)

        pltpu.sync_copy(cnt_hbm.at[wid], cb)
        total = cb[pl.ds(0, 16)][0]
        ng = total // 512

        def load_lists(g, p):
            gb = g * 512
            return (pltpu.async_copy(slist_hbm.at[wid, pl.ds(gb, 512)],
                                     sbuf.at[p], lsem[p][0]),
                    pltpu.async_copy(dlist_hbm.at[wid, pl.ds(gb, 512)],
                                     dbuf.at[p], lsem[p][1]))

        nrb = 2

        def gather_rows(p, hl):
            hl[0].wait()
            hl[1].wait()
            return [pltpu.async_copy(
                        a_hbm.at[sbuf.at[p, pl.ds(q * 128, 128)]],
                        rows_v.at[p % nrb, q], sems[p][q])
                    for q in range(4)]

        def rmw_group(p, hg):
            for q in range(4):
                hg[q].wait()

                def rmw(g8, _):
                    basev = dbuf[p, pl.ds(q * 128 + g8 * 16, 16)] * d
                    for b in range(16):
                        ab = basev[b]
                        for j in range(nvr):
                            cur = acc[pl.ds(ab + j * 16, 16)]
                            acc[pl.ds(ab + j * 16, 16)] = jnp.maximum(
                                cur,
                                rows_v[p % nrb, q,
                                       g8 * 16 + b, pl.ds(j * 16, 16)])
                    return 0

                lax.fori_loop(0, 8, rmw, 0)

        def pair(kk, _):
            g0 = kk * 2
            hl0 = load_lists(g0, 0)

            @pl.when(g0 + 1 < ng)
            def _():
                hl1 = load_lists(g0 + 1, 1)
                if nrb == 2:
                    hg0 = gather_rows(0, hl0)
                    hg1 = gather_rows(1, hl1)
                    rmw_group(0, hg0)
                    rmw_group(1, hg1)
                else:
                    rmw_group(0, gather_rows(0, hl0))
                    rmw_group(1, gather_rows(1, hl1))

            @pl.when(g0 + 1 >= ng)
            def _():
                rmw_group(0, gather_rows(0, hl0))
            return 0

        lax.fori_loop(0, (ng + 1) // 2, pair, 0)

        pltpu.sync_copy(acc.at[pl.ds(0, RANGE * d)],
                        out_hbm.at[c, pl.ds(lo * d, RANGE * d)])

    return seg_max


_segment_max_64 = _make_segment_max(64)
_segment_max_32 = _make_segment_max(32)


_BE = 1280


def _mlp_body(ea_ref, xg_ref, w1, b1, w2, b2, w3, b3, w4, b4, w5, b5,
              out_ref):
    f32 = jnp.float32
    h = jnp.maximum(jnp.dot(ea_ref[...], w1[...],
                            preferred_element_type=f32) + b1[...], 0.0)
    h = jnp.maximum(jnp.dot(h, w2[...],
                            preferred_element_type=f32) + b2[...], 0.0)
    h = jnp.maximum(jnp.dot(h, w3[...],
                            preferred_element_type=f32) + b3[...], 0.0)
    h = jnp.maximum(jnp.dot(h, w4[...],
                            preferred_element_type=f32) + b4[...], 0.0)
    w = jax.nn.sigmoid(jnp.dot(h, w5[...],
                               preferred_element_type=f32) + b5[...])
    acc = xg_ref[:, 0:1] * w[:, 0:128]
    for i in range(1, 7):
        acc = acc + xg_ref[:, i:i + 1] * w[:, i * 128:(i + 1) * 128]
    out_ref[...] = acc


def _mlp_msg(ea, xg, p):
    ws = []
    for i in range(1, 6):
        ws.append(p[f"mlp_W{i}"])
        ws.append(p[f"mlp_b{i}"].reshape(1, -1))
    full = lambda a: pl.BlockSpec(a.shape, lambda i: (0,) * a.ndim)
    return pl.pallas_call(
        _mlp_body,
        grid=(E // _BE,),
        in_specs=[
            pl.BlockSpec((_BE, 3), lambda i: (i, 0)),
            pl.BlockSpec((_BE, 16), lambda i: (i, 0)),
        ] + [full(a) for a in ws],
        out_specs=pl.BlockSpec((_BE, 128), lambda i: (i, 0)),
        out_shape=jax.ShapeDtypeStruct((E, 128), jnp.float32),
    )(ea, xg, *ws)


_BN = 2048


def _combine_mean_ab(aggp, degp, bias, wts, wpms, cb):
    d = wts.shape[1]

    def body(aggp_ref, degp_ref, bias_ref, wts_ref, wpms_ref, cb_ref,
             a_ref, b_ref, degc_ref):
        agg = aggp_ref[0] + aggp_ref[1]
        deg = degp_ref[0, :, 0:1] + degp_ref[1, :, 0:1]
        h = agg / jnp.maximum(deg, 1.0) + bias_ref[...]
        a_ref[...] = jnp.dot(h, wts_ref[...], precision=jax.lax.Precision.HIGHEST,
                             preferred_element_type=jnp.float32)
        b_ref[...] = jnp.dot(h, wpms_ref[...], precision=jax.lax.Precision.HIGHEST,
                             preferred_element_type=jnp.float32) + cb_ref[...]
        degc_ref[...] = jnp.broadcast_to(deg, (_BN, 16))

    full = lambda a: pl.BlockSpec(a.shape, lambda i: (0,) * a.ndim)
    return pl.pallas_call(
        body,
        grid=(N_PAD // _BN,),
        in_specs=[
            pl.BlockSpec((2, _BN, 128), lambda i: (0, i, 0)),
            pl.BlockSpec((2, _BN, 16), lambda i: (0, i, 0)),
            full(bias), full(wts), full(wpms), full(cb),
        ],
        out_specs=[
            pl.BlockSpec((_BN, d), lambda i: (i, 0)),
            pl.BlockSpec((_BN, d), lambda i: (i, 0)),
            pl.BlockSpec((_BN, 16), lambda i: (i, 0)),
        ],
        out_shape=[
            jax.ShapeDtypeStruct((N_PAD, d), jnp.float32),
            jax.ShapeDtypeStruct((N_PAD, d), jnp.float32),
            jax.ShapeDtypeStruct((N_PAD, 16), jnp.float32),
        ],
    )(aggp, degp, bias, wts, wpms, cb)


def _next_layer_ab(mp, bprev, degc, wts, wpms, cb):
    dp = bprev.shape[1]
    d = wts.shape[1]

    def body(mp_ref, bprev_ref, degc_ref, wts_ref, wpms_ref, cb_ref,
             a_ref, b_ref):
        m = jnp.maximum(mp_ref[0], mp_ref[1])
        h = jnp.where(degc_ref[:, 0:1] > 0.5, bprev_ref[...] + m, 0.0)
        a_ref[...] = jnp.dot(h, wts_ref[...], precision=jax.lax.Precision.HIGHEST,
                             preferred_element_type=jnp.float32)
        b_ref[...] = jnp.dot(h, wpms_ref[...], precision=jax.lax.Precision.HIGHEST,
                             preferred_element_type=jnp.float32) + cb_ref[...]

    full = lambda a: pl.BlockSpec(a.shape, lambda i: (0,) * a.ndim)
    return pl.pallas_call(
        body,
        grid=(N_PAD // _BN,),
        in_specs=[
            pl.BlockSpec((2, _BN, dp), lambda i: (0, i, 0)),
            pl.BlockSpec((_BN, dp), lambda i: (i, 0)),
            pl.BlockSpec((_BN, 16), lambda i: (i, 0)),
            full(wts), full(wpms), full(cb),
        ],
        out_specs=[
            pl.BlockSpec((_BN, d), lambda i: (i, 0)),
            pl.BlockSpec((_BN, d), lambda i: (i, 0)),
        ],
        out_shape=[
            jax.ShapeDtypeStruct((N_PAD, d), jnp.float32),
            jax.ShapeDtypeStruct((N_PAD, d), jnp.float32),
        ],
    )(mp, bprev, degc, wts, wpms, cb)


def _final_h(mp, bprev, degc):
    dp = bprev.shape[1]

    def body(mp_ref, bprev_ref, degc_ref, h_ref):
        m = jnp.maximum(mp_ref[0], mp_ref[1])
        h_ref[...] = jnp.where(degc_ref[:, 0:1] > 0.5, bprev_ref[...] + m,
                               0.0)

    return pl.pallas_call(
        body,
        grid=(N_PAD // _BN,),
        in_specs=[
            pl.BlockSpec((2, _BN, dp), lambda i: (0, i, 0)),
            pl.BlockSpec((_BN, dp), lambda i: (i, 0)),
            pl.BlockSpec((_BN, 16), lambda i: (i, 0)),
        ],
        out_specs=pl.BlockSpec((_BN, dp), lambda i: (i, 0)),
        out_shape=jax.ShapeDtypeStruct((N_PAD, dp), jnp.float32),
    )(mp, bprev, degc)


def kernel(x, edge_index, edge_attr, params):
    p = params
    src = edge_index[0].astype(jnp.int32)
    dst = edge_index[1].astype(jnp.int32)
    xpad = jnp.pad(x, ((0, 0), (0, 16 - x.shape[1])))

    wts, wpms, cbs = [], [], []
    for i in (1, 2, 3):
        scale = p[f"ec{i}_g"] / jnp.sqrt(p[f"ec{i}_rv"] + 1e-5)
        shift = p[f"ec{i}_b"] - p[f"ec{i}_rm"] * scale
        wts.append(p[f"ec{i}_Wt"] * scale[None, :])
        wpms.append((p[f"ec{i}_Wp"] - p[f"ec{i}_Wt"]) * scale[None, :])
        cbs.append(((p[f"ec{i}_bt"] + p[f"ec{i}_bp"]) * scale
                    + shift).reshape(1, -1))

    xg = _gather_x(src, xpad)
    slist, dlist, cnt = _bucketize(dst, src)
    msg = _mlp_msg(edge_attr, xg, p)
    aggp, degp = _segment_sum(dst, msg)
    a1, b1, degc = _combine_mean_ab(
        aggp, degp, p["nnconv_bias"].reshape(1, -1), wts[0], wpms[0], cbs[0])
    mp1 = _segment_max_64(slist, dlist, cnt, a1).reshape(2, N_PAD, 64)
    a2, b2 = _next_layer_ab(mp1, b1, degc, wts[1], wpms[1], cbs[1])
    mp2 = _segment_max_32(slist, dlist, cnt, a2).reshape(2, N_PAD, 32)
    a3, b3 = _next_layer_ab(mp2, b2, degc, wts[2], wpms[2], cbs[2])
    mp3 = _segment_max_32(slist, dlist, cnt, a3).reshape(2, N_PAD, 32)
    h = _final_h(mp3, b3, degc)
    return h[:N]
